# Initial kernel scaffold; baseline (speedup 1.0000x reference)
import jax
import jax.numpy as jnp
from jax.experimental import pallas as pl


def kernel(class_log_probabilities, last_predictions, last_log_probabilities):
    raise NotImplementedError("write your pallas kernel here")


# speedup vs baseline: 3.5373x; 3.5373x over previous
import functools

import jax
import jax.numpy as jnp
from jax import lax
from jax.experimental import pallas as pl

_BEAM = 4
_EOS = 0
_NEG_INF = float("-inf")


def _scan_body(x_ref, topv_ref, topi_ref, lse_ref):
    x = x_ref[...]
    rows, C = x.shape
    m = jnp.max(x, axis=1, keepdims=True)
    s = jnp.sum(jnp.exp(x - m), axis=1, keepdims=True)
    lse_ref[...] = m + jnp.log(s)

    iota = lax.broadcasted_iota(jnp.int32, x.shape, 1)
    xc = x
    vals = []
    idxs = []
    for _ in range(_BEAM):
        mk = jnp.max(xc, axis=1, keepdims=True)
        ik = jnp.min(jnp.where(xc == mk, iota, C), axis=1, keepdims=True)
        vals.append(mk)
        idxs.append(ik)
        xc = jnp.where(iota == ik, _NEG_INF, xc)
    topv_ref[...] = jnp.concatenate(vals, axis=1)
    topi_ref[...] = jnp.concatenate(idxs, axis=1)


def _merge_body(C, topv_ref, topi_ref, lse16_ref, lp16_ref, pred16_ref,
                outv_ref, outc_ref, outb_ref):
    topv = topv_ref[...]
    topi = topi_ref[...]
    lse16 = lse16_ref[...]
    lp16 = lp16_ref[...]
    pred16 = pred16_ref[...]
    B = topv.shape[0]

    lane = lax.broadcasted_iota(jnp.int32, (B, 16), 1)
    beam = lane // _BEAM
    k_in_beam = lane % _BEAM
    finished = pred16 == _EOS

    base_v = lp16 + topv - lse16
    fin_v = jnp.where(k_in_beam == 0, lp16, _NEG_INF)
    cand_v = jnp.where(finished, fin_v, base_v)
    cand_c = jnp.where(finished, 0, topi)
    flat = beam * C + cand_c

    big = _BEAM * C
    outv = []
    outc = []
    outb = []
    for _ in range(_BEAM):
        mk = jnp.max(cand_v, axis=1, keepdims=True)
        fsel = jnp.min(jnp.where(cand_v == mk, flat, big), axis=1,
                       keepdims=True)
        outv.append(mk)
        outb.append(fsel // C)
        outc.append(fsel % C)
        cand_v = jnp.where(flat == fsel, _NEG_INF, cand_v)
    outv_ref[...] = jnp.concatenate(outv, axis=1)
    outc_ref[...] = jnp.concatenate(outc, axis=1)
    outb_ref[...] = jnp.concatenate(outb, axis=1)


@jax.jit
def kernel(class_log_probabilities, last_predictions, last_log_probabilities):
    R, C = class_log_probabilities.shape
    B, beam = last_log_probabilities.shape
    ROWS_PER_STEP = 64
    steps = R // ROWS_PER_STEP

    topv, topi, lse = pl.pallas_call(
        _scan_body,
        grid=(steps,),
        in_specs=[pl.BlockSpec((ROWS_PER_STEP, C), lambda i: (i, 0))],
        out_specs=[
            pl.BlockSpec((ROWS_PER_STEP, beam), lambda i: (i, 0)),
            pl.BlockSpec((ROWS_PER_STEP, beam), lambda i: (i, 0)),
            pl.BlockSpec((ROWS_PER_STEP, 1), lambda i: (i, 0)),
        ],
        out_shape=[
            jax.ShapeDtypeStruct((R, beam), jnp.float32),
            jax.ShapeDtypeStruct((R, beam), jnp.int32),
            jax.ShapeDtypeStruct((R, 1), jnp.float32),
        ],
    )(class_log_probabilities)

    topv16 = topv.reshape(B, beam * _BEAM)
    topi16 = topi.reshape(B, beam * _BEAM)
    lse16 = jnp.repeat(lse.reshape(B, beam), _BEAM, axis=1)
    lp16 = jnp.repeat(last_log_probabilities, _BEAM, axis=1)
    pred16 = jnp.repeat(last_predictions.reshape(B, beam), _BEAM, axis=1)

    outv, outc, outb = pl.pallas_call(
        functools.partial(_merge_body, C),
        out_shape=[
            jax.ShapeDtypeStruct((B, beam), jnp.float32),
            jax.ShapeDtypeStruct((B, beam), jnp.int32),
            jax.ShapeDtypeStruct((B, beam), jnp.int32),
        ],
    )(topv16, topi16, lse16, lp16, pred16)
    return outv, outc, outb
